# Initial kernel scaffold; baseline (speedup 1.0000x reference)
import jax
import jax.numpy as jnp
from jax.experimental import pallas as pl


def kernel(feat0, feat1, feat2, nid, edge_r0, edge_r1, emb0, emb1, emb2, W_self_r0, W_neigh_r0, b_r0, W_self_r1, W_neigh_r1, b_r1, h_bias, W_cls, b_cls):
    raise NotImplementedError("write your pallas kernel here")


# speedup vs baseline: 5.1551x; 5.1551x over previous
import functools

import jax
import jax.numpy as jnp
from jax import lax
from jax.experimental import pallas as pl
from jax.experimental.pallas import tpu as pltpu
from jax.experimental.pallas import tpu_sc as plsc

N = 10000
D = 128
OUT = 16
E = 160000
V = 1000

NC = 2
NS = 16
NW = NC * NS

NP = 10240
NPW = NP // NW
ACH = 80
NROW = NP // NS

CK = 128
ETP = 10112
NCH = ETP // CK
EPAD = ETP * NS

_mesh = plsc.VectorSubcoreMesh(core_axis_name="c", subcore_axis_name="s")
_sc_params = pltpu.CompilerParams(use_tc_tiling_on_sc=False)


def _build_h_body(f0_hbm, f1_hbm, f2_hbm, e0_hbm, e1_hbm, e2_hbm, h_hbm,
                  fidx0, fidx1, fidx2, b0, b1, b2, hbuf):
  c = lax.axis_index("c")
  s = lax.axis_index("s")
  wid = s * NC + c
  nbase = wid * NPW
  for f_hbm, fidx in ((f0_hbm, fidx0), (f1_hbm, fidx1), (f2_hbm, fidx2)):
    pltpu.sync_copy(f_hbm.at[pl.ds(nbase, NPW)], fidx)
  for cc in range(NPW // ACH):
    off = cc * ACH
    pltpu.sync_copy(e0_hbm.at[fidx0.at[pl.ds(off, ACH)]], b0)
    pltpu.sync_copy(e1_hbm.at[fidx1.at[pl.ds(off, ACH)]], b1)
    pltpu.sync_copy(e2_hbm.at[fidx2.at[pl.ds(off, ACH)]], b2)

    @pl.loop(0, ACH)
    def _(i):
      for j in range(D // 16):
        sl = pl.ds(j * 16, 16)
        hbuf[i, sl] = b0[i, sl] + b1[i, sl] + b2[i, sl]

    pltpu.sync_copy(hbuf, h_hbm.at[pl.ds(nbase + off, ACH), :])


_build_h = pl.kernel(
    _build_h_body,
    out_type=jax.ShapeDtypeStruct((NP, D), jnp.float32),
    mesh=_mesh,
    compiler_params=_sc_params,
    scratch_types=[
        pltpu.VMEM((NPW,), jnp.int32),
        pltpu.VMEM((NPW,), jnp.int32),
        pltpu.VMEM((NPW,), jnp.int32),
        pltpu.VMEM((ACH, D), jnp.float32),
        pltpu.VMEM((ACH, D), jnp.float32),
        pltpu.VMEM((ACH, D), jnp.float32),
        pltpu.VMEM((ACH, D), jnp.float32),
    ],
)


def _mp_one_relation(t, h_hbm, src_hbm, dst_hbm, hn_hbm, deg_hbm,
                     acc, dacc, rows, dbuf, ones, sidx, didx):
  row0 = t * NROW
  base0 = t * ETP

  @pl.loop(0, NCH)
  def _(k):
    base = base0 + k * CK
    pltpu.sync_copy(src_hbm.at[pl.ds(base, CK)], sidx)
    pltpu.sync_copy(dst_hbm.at[pl.ds(base, CK)], didx)
    si = plsc.Indices(sidx, ignored_value=-1)
    di = plsc.Indices(didx, ignored_value=-1)
    pltpu.sync_copy(h_hbm.at[si], rows)
    pltpu.sync_copy(rows, acc.at[di], add=True)
    pltpu.sync_copy(ones, dacc.at[di], add=True)

  plsc.subcore_barrier()

  for j in range(NROW // CK):
    pltpu.sync_copy(acc.at[pl.ds(row0 + j * CK, CK), :], rows)
    pltpu.sync_copy(rows, hn_hbm.at[pl.ds(row0 + j * CK, CK), :])
  pltpu.sync_copy(dacc.at[pl.ds(row0, NROW), :], dbuf)
  pltpu.sync_copy(dbuf, deg_hbm.at[pl.ds(row0, NROW), :])


def _msgpass_body(h_hbm, src0_hbm, dst0_hbm, src1_hbm, dst1_hbm,
                  hn0_hbm, deg0_hbm, hn1_hbm, deg1_hbm,
                  acc, dacc, rows, dbuf, ones, sidx, didx):
  c = lax.axis_index("c")
  t = lax.axis_index("s")
  row0 = t * NROW

  @pl.loop(0, CK)
  def _(i):
    z16 = jnp.zeros((16,), jnp.float32)
    for j in range(D // 16):
      rows[i, pl.ds(j * 16, 16)] = z16
    ones[i, :] = jnp.ones((16,), jnp.float32)

  @pl.loop(0, NROW)
  def _(i):
    dbuf[i, :] = jnp.zeros((16,), jnp.float32)

  for j in range(NROW // CK):
    pltpu.sync_copy(rows, acc.at[pl.ds(row0 + j * CK, CK), :])
  pltpu.sync_copy(dbuf, dacc.at[pl.ds(row0, NROW), :])
  plsc.subcore_barrier()

  @pl.when(c == 0)
  def _():
    _mp_one_relation(t, h_hbm, src0_hbm, dst0_hbm, hn0_hbm, deg0_hbm,
                     acc, dacc, rows, dbuf, ones, sidx, didx)

  @pl.when(c == 1)
  def _():
    _mp_one_relation(t, h_hbm, src1_hbm, dst1_hbm, hn1_hbm, deg1_hbm,
                     acc, dacc, rows, dbuf, ones, sidx, didx)


_msgpass = pl.kernel(
    _msgpass_body,
    out_type=(
        jax.ShapeDtypeStruct((NP, D), jnp.float32),
        jax.ShapeDtypeStruct((NP, 16), jnp.float32),
        jax.ShapeDtypeStruct((NP, D), jnp.float32),
        jax.ShapeDtypeStruct((NP, 16), jnp.float32),
    ),
    mesh=_mesh,
    compiler_params=_sc_params,
    scratch_types=[
        pltpu.VMEM_SHARED((NP, D), jnp.float32),
        pltpu.VMEM_SHARED((NP, 16), jnp.float32),
        pltpu.VMEM((CK, D), jnp.float32),
        pltpu.VMEM((NROW, 16), jnp.float32),
        pltpu.VMEM((CK, 16), jnp.float32),
        pltpu.VMEM((CK,), jnp.int32),
        pltpu.VMEM((CK,), jnp.int32),
    ],
)

BN = 512


def _dense_body(h_ref, hn0_ref, hn1_ref, d0_ref, d1_ref,
                ws0_ref, ws1_ref, wn0_ref, wn1_ref,
                b0_ref, b1_ref, hb_ref, wc_ref, bc_ref,
                hout_ref, oout_ref):
  x = h_ref[...]
  r0 = 1.0 / jnp.maximum(d0_ref[...][:, :1], 1.0)
  r1 = 1.0 / jnp.maximum(d1_ref[...][:, :1], 1.0)
  acc = jnp.dot(x, ws0_ref[...] + ws1_ref[...],
                preferred_element_type=jnp.float32)
  acc += jnp.dot(hn0_ref[...] * r0, wn0_ref[...],
                 preferred_element_type=jnp.float32)
  acc += jnp.dot(hn1_ref[...] * r1, wn1_ref[...],
                 preferred_element_type=jnp.float32)
  h = jnp.maximum(acc + b0_ref[...] + b1_ref[...] + hb_ref[...], 0.0)
  hout_ref[...] = h
  oout_ref[...] = jnp.dot(h, wc_ref[...],
                          preferred_element_type=jnp.float32) + bc_ref[...]


def _dense(h, hn0, hn1, deg0, deg1, ws0, ws1, wn0, wn1, b0, b1, hb, wc, bc):
  nblk = NP // BN
  full = lambda i: (0, 0)
  return pl.pallas_call(
      _dense_body,
      grid=(nblk,),
      in_specs=[
          pl.BlockSpec((BN, D), lambda i: (i, 0)),
          pl.BlockSpec((BN, D), lambda i: (i, 0)),
          pl.BlockSpec((BN, D), lambda i: (i, 0)),
          pl.BlockSpec((BN, 16), lambda i: (i, 0)),
          pl.BlockSpec((BN, 16), lambda i: (i, 0)),
          pl.BlockSpec((D, D), full),
          pl.BlockSpec((D, D), full),
          pl.BlockSpec((D, D), full),
          pl.BlockSpec((D, D), full),
          pl.BlockSpec((1, D), full),
          pl.BlockSpec((1, D), full),
          pl.BlockSpec((1, D), full),
          pl.BlockSpec((D, OUT), full),
          pl.BlockSpec((1, OUT), full),
      ],
      out_specs=[
          pl.BlockSpec((BN, D), lambda i: (i, 0)),
          pl.BlockSpec((BN, OUT), lambda i: (i, 0)),
      ],
      out_shape=[
          jax.ShapeDtypeStruct((NP, D), jnp.float32),
          jax.ShapeDtypeStruct((NP, OUT), jnp.float32),
      ],
  )(h, hn0, hn1, deg0, deg1, ws0, ws1, wn0, wn1, b0, b1, hb, wc, bc)


def kernel(feat0, feat1, feat2, nid, edge_r0, edge_r1, emb0, emb1, emb2,
           W_self_r0, W_neigh_r0, b_r0, W_self_r1, W_neigh_r1, b_r1,
           h_bias, W_cls, b_cls):
  del nid
  i32 = jnp.int32
  zpadn = jnp.zeros((NP - N,), i32)
  f0 = jnp.concatenate([feat0.astype(i32), zpadn])
  f1 = jnp.concatenate([feat1.astype(i32), zpadn])
  f2 = jnp.concatenate([feat2.astype(i32), zpadn])
  epad = jnp.full((EPAD - E,), -1, i32)
  src0 = jnp.concatenate([edge_r0[0].astype(i32), epad])
  dst0 = jnp.concatenate([edge_r0[1].astype(i32), epad])
  src1 = jnp.concatenate([edge_r1[0].astype(i32), epad])
  dst1 = jnp.concatenate([edge_r1[1].astype(i32), epad])

  h_nodes = _build_h(f0, f1, f2, emb0, emb1, emb2)
  hn0, deg0, hn1, deg1 = _msgpass(h_nodes, src0, dst0, src1, dst1)
  h_full, out_full = _dense(
      h_nodes, hn0, hn1, deg0, deg1,
      W_self_r0, W_self_r1, W_neigh_r0, W_neigh_r1,
      b_r0.reshape(1, D), b_r1.reshape(1, D), h_bias.reshape(1, D),
      W_cls, b_cls.reshape(1, OUT))
  return (out_full[:N], h_full[:N])


# speedup vs baseline: 9.5286x; 1.8484x over previous
import functools

import jax
import jax.numpy as jnp
from jax import lax
from jax.experimental import pallas as pl
from jax.experimental.pallas import tpu as pltpu
from jax.experimental.pallas import tpu_sc as plsc

N = 10000
D = 128
OUT = 16
E = 160000
V = 1000

NC = 2
NS = 16
NW = NC * NS

NP = 10240
NPW = NP // NW
ACH = 80
NROW = NP // NS

CK = 128
NCH = 80
ETP = NCH * CK
EPAD = ETP * NS

_mesh = plsc.VectorSubcoreMesh(core_axis_name="c", subcore_axis_name="s")
_sc_params = pltpu.CompilerParams(use_tc_tiling_on_sc=False)


_NACH = NPW // ACH


def _build_h_body(f0_hbm, f1_hbm, f2_hbm, e0_hbm, e1_hbm, e2_hbm, h_hbm,
                  fidx0, fidx1, fidx2,
                  b00, b10, b20, b01, b11, b21, hbuf0, hbuf1,
                  gsem0, gsem1, wsem0, wsem1):
  c = lax.axis_index("c")
  s = lax.axis_index("s")
  wid = s * NC + c
  nbase = wid * NPW
  for f_hbm, fidx in ((f0_hbm, fidx0), (f1_hbm, fidx1), (f2_hbm, fidx2)):
    pltpu.sync_copy(f_hbm.at[pl.ds(nbase, NPW)], fidx)

  bufs = ((b00, b10, b20), (b01, b11, b21))
  hbufs = (hbuf0, hbuf1)
  gsems = (gsem0, gsem1)
  wsems = (wsem0, wsem1)

  def gathers(cc, bs):
    off = cc * ACH
    return [
        pltpu.make_async_copy(e_hbm.at[fidx.at[pl.ds(off, ACH)]],
                              bufs[bs][j], gsems[bs])
        for j, (e_hbm, fidx) in enumerate(
            ((e0_hbm, fidx0), (e1_hbm, fidx1), (e2_hbm, fidx2)))
    ]

  def write_out(cc, bs):
    return pltpu.make_async_copy(
        hbufs[bs], h_hbm.at[pl.ds(nbase + cc * ACH, ACH), :], wsems[bs])

  for d in gathers(0, 0):
    d.start()
  for cc in range(_NACH):
    bs = cc % 2
    if cc + 1 < _NACH:
      for d in gathers(cc + 1, 1 - bs):
        d.start()
    for d in gathers(cc, bs):
      d.wait()
    if cc >= 2:
      write_out(cc - 2, bs).wait()
    b0, b1, b2 = bufs[bs]
    hbuf = hbufs[bs]

    @pl.loop(0, ACH)
    def _(i):
      for j in range(D // 16):
        sl = pl.ds(j * 16, 16)
        hbuf[i, sl] = b0[i, sl] + b1[i, sl] + b2[i, sl]

    write_out(cc, bs).start()
  write_out(_NACH - 2, 0).wait()
  write_out(_NACH - 1, 1).wait()


_build_h = pl.kernel(
    _build_h_body,
    out_type=jax.ShapeDtypeStruct((NP, D), jnp.float32),
    mesh=_mesh,
    compiler_params=_sc_params,
    scratch_types=[
        pltpu.VMEM((NPW,), jnp.int32),
        pltpu.VMEM((NPW,), jnp.int32),
        pltpu.VMEM((NPW,), jnp.int32),
        pltpu.VMEM((ACH, D), jnp.float32),
        pltpu.VMEM((ACH, D), jnp.float32),
        pltpu.VMEM((ACH, D), jnp.float32),
        pltpu.VMEM((ACH, D), jnp.float32),
        pltpu.VMEM((ACH, D), jnp.float32),
        pltpu.VMEM((ACH, D), jnp.float32),
        pltpu.VMEM((ACH, D), jnp.float32),
        pltpu.VMEM((ACH, D), jnp.float32),
        pltpu.SemaphoreType.DMA,
        pltpu.SemaphoreType.DMA,
        pltpu.SemaphoreType.DMA,
        pltpu.SemaphoreType.DMA,
    ],
)


def _mp_one_relation(t, h_hbm, src_hbm, dst_hbm, hn_hbm, deg_hbm,
                     acc, dacc, rows0, rows1, zbuf, ones, sidx, didx,
                     gsem0, gsem1, isems):
  row0 = t * NROW
  cbase = t * NCH
  rows = (rows0, rows1)
  gsem = (gsem0, gsem1)

  def idx_copies(k, slot):
    sem = isems[slot]
    return (
        pltpu.make_async_copy(src_hbm.at[cbase + k], sidx.at[slot], sem),
        pltpu.make_async_copy(dst_hbm.at[cbase + k], didx.at[slot], sem),
    )

  def gather_copy(slot, b):
    si = plsc.Indices(sidx.at[slot], ignored_value=-1)
    return pltpu.make_async_copy(h_hbm.at[si], rows[b], gsem[b])

  for k in range(2):
    for d in idx_copies(k, k):
      d.start()
      d.wait()
  for k in range(2, 4):
    for d in idx_copies(k, k):
      d.start()
  gather_copy(0, 0).start()
  gather_copy(1, 1).start()

  @pl.loop(0, NCH // 4)
  def _(g):
    for q in range(4):
      k = 4 * g + q
      b = q % 2
      nslot = (q + 2) % 4
      gather_copy(q, b).wait()
      di = plsc.Indices(didx.at[q], ignored_value=-1)
      pltpu.sync_copy(rows[b], acc.at[di], add=True)
      pltpu.sync_copy(ones, dacc.at[di], add=True)

      @pl.when(k + 2 < NCH)
      def _():
        for d in idx_copies(k + 2, nslot):
          d.wait()
        gather_copy(nslot, b).start()

      @pl.when(k + 4 < NCH)
      def _():
        for d in idx_copies(k + 4, q):
          d.start()

  plsc.subcore_barrier()

  pltpu.sync_copy(acc.at[pl.ds(row0, NROW), :],
                  hn_hbm.at[pl.ds(row0, NROW), :])
  pltpu.sync_copy(dacc.at[pl.ds(row0, NROW), :],
                  deg_hbm.at[pl.ds(row0, NROW), :])


def _msgpass_body(h_hbm, src0_hbm, dst0_hbm, src1_hbm, dst1_hbm,
                  hn0_hbm, deg0_hbm, hn1_hbm, deg1_hbm,
                  acc, dacc, rows0, rows1, zbuf, ones, sidx, didx,
                  gsem0, gsem1, isem0, isem1, isem2, isem3):
  c = lax.axis_index("c")
  t = lax.axis_index("s")
  row0 = t * NROW
  isems = (isem0, isem1, isem2, isem3)

  @pl.loop(0, CK)
  def _(i):
    z16 = jnp.zeros((16,), jnp.float32)
    for j in range(D // 16):
      rows0[i, pl.ds(j * 16, 16)] = z16
    ones[i, :] = jnp.ones((16,), jnp.float32)
    zbuf[i, :] = z16

  for j in range(NROW // CK):
    pltpu.sync_copy(rows0, acc.at[pl.ds(row0 + j * CK, CK), :])
    pltpu.sync_copy(zbuf, dacc.at[pl.ds(row0 + j * CK, CK), :])
  plsc.subcore_barrier()

  @pl.when(c == 0)
  def _():
    _mp_one_relation(t, h_hbm, src0_hbm, dst0_hbm, hn0_hbm, deg0_hbm,
                     acc, dacc, rows0, rows1, zbuf, ones, sidx, didx,
                     gsem0, gsem1, isems)

  @pl.when(c == 1)
  def _():
    _mp_one_relation(t, h_hbm, src1_hbm, dst1_hbm, hn1_hbm, deg1_hbm,
                     acc, dacc, rows0, rows1, zbuf, ones, sidx, didx,
                     gsem0, gsem1, isems)


_msgpass = pl.kernel(
    _msgpass_body,
    out_type=(
        jax.ShapeDtypeStruct((NP, D), jnp.float32),
        jax.ShapeDtypeStruct((NP, 16), jnp.float32),
        jax.ShapeDtypeStruct((NP, D), jnp.float32),
        jax.ShapeDtypeStruct((NP, 16), jnp.float32),
    ),
    mesh=_mesh,
    compiler_params=_sc_params,
    scratch_types=[
        pltpu.VMEM_SHARED((NP, D), jnp.float32),
        pltpu.VMEM_SHARED((NP, 16), jnp.float32),
        pltpu.VMEM((CK, D), jnp.float32),
        pltpu.VMEM((CK, D), jnp.float32),
        pltpu.VMEM((CK, 16), jnp.float32),
        pltpu.VMEM((CK, 16), jnp.float32),
        pltpu.VMEM((4, CK), jnp.int32),
        pltpu.VMEM((4, CK), jnp.int32),
        pltpu.SemaphoreType.DMA,
        pltpu.SemaphoreType.DMA,
        pltpu.SemaphoreType.DMA,
        pltpu.SemaphoreType.DMA,
        pltpu.SemaphoreType.DMA,
        pltpu.SemaphoreType.DMA,
    ],
)

BN = 512


def _dense_body(h_ref, hn0_ref, hn1_ref, d0_ref, d1_ref,
                ws0_ref, ws1_ref, wn0_ref, wn1_ref,
                b0_ref, b1_ref, hb_ref, wc_ref, bc_ref,
                hout_ref, oout_ref):
  x = h_ref[...]
  r0 = 1.0 / jnp.maximum(d0_ref[...][:, :1], 1.0)
  r1 = 1.0 / jnp.maximum(d1_ref[...][:, :1], 1.0)
  acc = jnp.dot(x, ws0_ref[...] + ws1_ref[...],
                preferred_element_type=jnp.float32)
  acc += jnp.dot(hn0_ref[...] * r0, wn0_ref[...],
                 preferred_element_type=jnp.float32)
  acc += jnp.dot(hn1_ref[...] * r1, wn1_ref[...],
                 preferred_element_type=jnp.float32)
  h = jnp.maximum(acc + b0_ref[...] + b1_ref[...] + hb_ref[...], 0.0)
  hout_ref[...] = h
  oout_ref[...] = jnp.dot(h, wc_ref[...],
                          preferred_element_type=jnp.float32) + bc_ref[...]


def _dense(h, hn0, hn1, deg0, deg1, ws0, ws1, wn0, wn1, b0, b1, hb, wc, bc):
  nblk = NP // BN
  full = lambda i: (0, 0)
  return pl.pallas_call(
      _dense_body,
      grid=(nblk,),
      in_specs=[
          pl.BlockSpec((BN, D), lambda i: (i, 0)),
          pl.BlockSpec((BN, D), lambda i: (i, 0)),
          pl.BlockSpec((BN, D), lambda i: (i, 0)),
          pl.BlockSpec((BN, 16), lambda i: (i, 0)),
          pl.BlockSpec((BN, 16), lambda i: (i, 0)),
          pl.BlockSpec((D, D), full),
          pl.BlockSpec((D, D), full),
          pl.BlockSpec((D, D), full),
          pl.BlockSpec((D, D), full),
          pl.BlockSpec((1, D), full),
          pl.BlockSpec((1, D), full),
          pl.BlockSpec((1, D), full),
          pl.BlockSpec((D, OUT), full),
          pl.BlockSpec((1, OUT), full),
      ],
      out_specs=[
          pl.BlockSpec((BN, D), lambda i: (i, 0)),
          pl.BlockSpec((BN, OUT), lambda i: (i, 0)),
      ],
      out_shape=[
          jax.ShapeDtypeStruct((NP, D), jnp.float32),
          jax.ShapeDtypeStruct((NP, OUT), jnp.float32),
      ],
  )(h, hn0, hn1, deg0, deg1, ws0, ws1, wn0, wn1, b0, b1, hb, wc, bc)


def kernel(feat0, feat1, feat2, nid, edge_r0, edge_r1, emb0, emb1, emb2,
           W_self_r0, W_neigh_r0, b_r0, W_self_r1, W_neigh_r1, b_r1,
           h_bias, W_cls, b_cls):
  del nid
  i32 = jnp.int32
  zpadn = jnp.zeros((NP - N,), i32)
  f0 = jnp.concatenate([feat0.astype(i32), zpadn])
  f1 = jnp.concatenate([feat1.astype(i32), zpadn])
  f2 = jnp.concatenate([feat2.astype(i32), zpadn])
  epad = jnp.full((EPAD - E,), -1, i32)
  ecols = lambda e: jnp.concatenate([e.astype(i32), epad]).reshape(NS * NCH, CK)
  src0 = ecols(edge_r0[0])
  dst0 = ecols(edge_r0[1])
  src1 = ecols(edge_r1[0])
  dst1 = ecols(edge_r1[1])

  h_nodes = _build_h(f0, f1, f2, emb0, emb1, emb2)
  hn0, deg0, hn1, deg1 = _msgpass(h_nodes, src0, dst0, src1, dst1)
  h_full, out_full = _dense(
      h_nodes, hn0, hn1, deg0, deg1,
      W_self_r0, W_self_r1, W_neigh_r0, W_neigh_r1,
      b_r0.reshape(1, D), b_r1.reshape(1, D), h_bias.reshape(1, D),
      W_cls, b_cls.reshape(1, OUT))
  return (out_full[:N], h_full[:N])
